# baseline (speedup 1.0000x reference)
import functools

import jax
import jax.numpy as jnp
from jax import lax
from jax.experimental import pallas as pl
from jax.experimental.pallas import tpu as pltpu
from jax.experimental.pallas import tpu_sc as plsc

N = 10000
E = 160000
D_IN = 256
D_HID = 512
G = 64
P_KEEP = 0.7

NC = 2
NS = 16
BLK = 128
E_PAD = 163840
EPT = E_PAD // NS
BPT = EPT // BLK
NACC = 10112
ZR = NACC // NS
DR = NACC // NS

R = 1000
NB = N // R


def _sc_mesh():
    return plsc.VectorSubcoreMesh(
        core_axis_name="c", subcore_axis_name="s", num_cores=NC, num_subcores=NS
    )


NBUF = 2
HB = 40


def _make_pass(z128_h, acc, idx2_v, didx2_v, bufs_v, sem_g, sem_s, r0, d0):
    def do_pass(tab_h, out_h, src2_h, dst2_h, blk_base, nblk):
        pltpu.sync_copy(z128_h.at[pl.ds(r0, ZR)], acc.at[pl.ds(r0, ZR)])

        def wait_g(b, j):
            pltpu.make_async_copy(tab_h.at[idx2_v.at[b]], bufs_v.at[j], sem_g).wait()

        def wait_s(j):
            pltpu.make_async_copy(
                bufs_v.at[j], acc.at[didx2_v.at[0]], sem_s).wait()

        for ph in range(nblk // HB):
            base = blk_base + ph * HB
            pltpu.sync_copy(src2_h.at[pl.ds(base, HB)], idx2_v)
            pltpu.sync_copy(dst2_h.at[pl.ds(base, HB)], didx2_v)
            if ph == 0:
                plsc.subcore_barrier()
            for k in range(NBUF):
                pltpu.async_copy(tab_h.at[idx2_v.at[k]], bufs_v.at[k], sem_g)

            def outer(g, carry):
                for j in range(NBUF):
                    b = g * NBUF + j
                    wait_g(b, j)
                    pltpu.async_copy(bufs_v.at[j], acc.at[didx2_v.at[b]], sem_s,
                                     add=True)

                    @pl.when(b > 0)
                    def _():
                        jp = (j - 1) % NBUF
                        wait_s(jp)

                        @pl.when(b - 1 + NBUF < HB)
                        def _():
                            pltpu.async_copy(
                                tab_h.at[idx2_v.at[b - 1 + NBUF]], bufs_v.at[jp],
                                sem_g)

                return carry

            lax.fori_loop(0, HB // NBUF, outer, 0)
            wait_s((HB - 1) % NBUF)
        plsc.subcore_barrier()
        pltpu.sync_copy(acc.at[pl.ds(d0, DR)], out_h.at[pl.ds(d0, DR)])
        plsc.subcore_barrier()

    return do_pass


_SC_SCRATCH = [
    pltpu.VMEM((HB, BLK), jnp.int32),
    pltpu.VMEM((HB, BLK), jnp.int32),
    pltpu.VMEM((NBUF, BLK, 128), jnp.float32),
    pltpu.VMEM_SHARED((NACC, 128), jnp.float32),
    pltpu.SemaphoreType.DMA,
    pltpu.SemaphoreType.DMA,
]


def _sc_aggregate_l1(xp0, xp1, mt128, src2, dst2, z128):
    @functools.partial(
        pl.kernel,
        out_type=[jax.ShapeDtypeStruct((NACC, 128), jnp.float32)] * 4,
        mesh=_sc_mesh(),
        scratch_types=_SC_SCRATCH,
    )
    def k(xp0_h, xp1_h, mt_h, src2_h, dst2_h, z128_h,
          sx0_h, sx1_h, dega_h, degb_h,
          idx2_v, didx2_v, bufs_v, acc, sem_g, sem_s):
        c = lax.axis_index("c")
        s = lax.axis_index("s")
        do_pass = _make_pass(z128_h, acc, idx2_v, didx2_v, bufs_v, sem_g, sem_s,
                             s * ZR, s * DR)
        halfb = (E_PAD // 2) // BLK

        @pl.when(c == 0)
        def _():
            do_pass(xp0_h, sx0_h, src2_h, dst2_h, s * BPT, BPT)
            do_pass(mt_h, dega_h, src2_h, dst2_h, s * (BPT // 2), BPT // 2)

        @pl.when(c == 1)
        def _():
            do_pass(xp1_h, sx1_h, src2_h, dst2_h, s * BPT, BPT)
            do_pass(mt_h, degb_h, src2_h, dst2_h, halfb + s * (BPT // 2), BPT // 2)

    return k(xp0, xp1, mt128, src2, dst2, z128)


def _sc_aggregate_l2(hp0, hp1, hp2, hp3, src2, dst2, z128):
    @functools.partial(
        pl.kernel,
        out_type=[jax.ShapeDtypeStruct((NACC, 128), jnp.float32)] * 4,
        mesh=_sc_mesh(),
        scratch_types=_SC_SCRATCH,
    )
    def k(hp0_h, hp1_h, hp2_h, hp3_h, src2_h, dst2_h, z128_h,
          sh0_h, sh1_h, sh2_h, sh3_h,
          idx2_v, didx2_v, bufs_v, acc, sem_g, sem_s):
        c = lax.axis_index("c")
        s = lax.axis_index("s")
        do_pass = _make_pass(z128_h, acc, idx2_v, didx2_v, bufs_v, sem_g, sem_s,
                             s * ZR, s * DR)

        @pl.when(c == 0)
        def _():
            do_pass(hp0_h, sh0_h, src2_h, dst2_h, s * BPT, BPT)
            do_pass(hp1_h, sh1_h, src2_h, dst2_h, s * BPT, BPT)

        @pl.when(c == 1)
        def _():
            do_pass(hp2_h, sh2_h, src2_h, dst2_h, s * BPT, BPT)
            do_pass(hp3_h, sh3_h, src2_h, dst2_h, s * BPT, BPT)

    return k(hp0, hp1, hp2, hp3, src2, dst2, z128)


def _prep(x, m2):
    def body(x_ref, m_ref, xp0_ref, xp1_ref, mt_ref):
        mcol = m_ref[...]
        xp = x_ref[...] * mcol
        xp0_ref[...] = xp[:, :128]
        xp1_ref[...] = xp[:, 128:]
        mt_ref[...] = jnp.broadcast_to(mcol, (R, 128))

    return pl.pallas_call(
        body,
        grid=(NB,),
        in_specs=[
            pl.BlockSpec((R, D_IN), lambda i: (i, 0)),
            pl.BlockSpec((R, 1), lambda i: (i, 0)),
        ],
        out_specs=[
            pl.BlockSpec((R, 128), lambda i: (i, 0)),
            pl.BlockSpec((R, 128), lambda i: (i, 0)),
            pl.BlockSpec((R, 128), lambda i: (i, 0)),
        ],
        out_shape=[
            jax.ShapeDtypeStruct((N, 128), jnp.float32),
            jax.ShapeDtypeStruct((N, 128), jnp.float32),
            jax.ShapeDtypeStruct((N, 128), jnp.float32),
        ],
    )(x, m2)


def _dense_l1(sx0, sx1, dega, degb, mt128, W1a, W1b, b1r):
    def body(sx0_ref, sx1_ref, dega_ref, degb_ref, mt_ref, wa_ref, wb_ref, b_ref,
             h_ref, hp0_ref, hp1_ref, hp2_ref, hp3_ref):
        mcol = mt_ref[:, 0:1]
        deg = dega_ref[:, 0:1] + degb_ref[:, 0:1]
        den = jnp.maximum(mcol * deg, 1.0)
        sc = mcol / den
        a0 = sx0_ref[...] * sc
        a1 = sx1_ref[...] * sc
        hh = (
            jnp.dot(a0, wa_ref[...], precision=lax.Precision.HIGHEST,
                    preferred_element_type=jnp.float32)
            + jnp.dot(a1, wb_ref[...], precision=lax.Precision.HIGHEST,
                      preferred_element_type=jnp.float32)
            + b_ref[...]
        )
        hh = jnp.maximum(hh, 0.0)
        h_ref[...] = hh
        hp = hh * mcol
        hp0_ref[...] = hp[:, 0:128]
        hp1_ref[...] = hp[:, 128:256]
        hp2_ref[...] = hp[:, 256:384]
        hp3_ref[...] = hp[:, 384:512]

    return pl.pallas_call(
        body,
        grid=(NB,),
        in_specs=[
            pl.BlockSpec((R, 128), lambda i: (i, 0)),
            pl.BlockSpec((R, 128), lambda i: (i, 0)),
            pl.BlockSpec((R, 128), lambda i: (i, 0)),
            pl.BlockSpec((R, 128), lambda i: (i, 0)),
            pl.BlockSpec((R, 128), lambda i: (i, 0)),
            pl.BlockSpec((128, D_HID), lambda i: (0, 0)),
            pl.BlockSpec((128, D_HID), lambda i: (0, 0)),
            pl.BlockSpec((1, D_HID), lambda i: (0, 0)),
        ],
        out_specs=[
            pl.BlockSpec((R, D_HID), lambda i: (i, 0)),
            pl.BlockSpec((R, 128), lambda i: (i, 0)),
            pl.BlockSpec((R, 128), lambda i: (i, 0)),
            pl.BlockSpec((R, 128), lambda i: (i, 0)),
            pl.BlockSpec((R, 128), lambda i: (i, 0)),
        ],
        out_shape=[
            jax.ShapeDtypeStruct((N, D_HID), jnp.float32),
            jax.ShapeDtypeStruct((N, 128), jnp.float32),
            jax.ShapeDtypeStruct((N, 128), jnp.float32),
            jax.ShapeDtypeStruct((N, 128), jnp.float32),
            jax.ShapeDtypeStruct((N, 128), jnp.float32),
        ],
    )(sx0, sx1, dega, degb, mt128, W1a, W1b, b1r)


def _dense_l2_pool(sh, dega, degb, mt128, pt, W2c, b2r):
    def body(sh0_ref, sh1_ref, sh2_ref, sh3_ref, dega_ref, degb_ref, mt_ref,
             pt_ref, w0_ref, w1_ref, w2_ref, w3_ref, b_ref, pooled_ref, cnt_ref):
        i = pl.program_id(0)

        @pl.when(i == 0)
        def _():
            pooled_ref[...] = jnp.zeros_like(pooled_ref)
            cnt_ref[...] = jnp.zeros_like(cnt_ref)

        mcol = mt_ref[:, 0:1]
        deg = dega_ref[:, 0:1] + degb_ref[:, 0:1]
        den = jnp.maximum(mcol * deg, 1.0)
        sc = mcol / den
        v = b_ref[...] + sum(
            jnp.dot(sref[...] * sc, wref[...], precision=lax.Precision.HIGHEST,
                    preferred_element_type=jnp.float32)
            for sref, wref in ((sh0_ref, w0_ref), (sh1_ref, w1_ref),
                               (sh2_ref, w2_ref), (sh3_ref, w3_ref))
        )
        v = jnp.maximum(v, 0.0)
        ptv = pt_ref[...]
        pooled_ref[...] += lax.dot_general(
            ptv, v, (((0,), (0,)), ((), ())),
            precision=lax.Precision.HIGHEST, preferred_element_type=jnp.float32)
        cnt_ref[...] += lax.dot_general(
            ptv, jnp.ones((R, 128), jnp.float32), (((0,), (0,)), ((), ())),
            precision=lax.Precision.HIGHEST, preferred_element_type=jnp.float32)

        @pl.when(i == NB - 1)
        def _():
            pooled_ref[...] = pooled_ref[...] / jnp.maximum(cnt_ref[:, 0:1], 1.0)

    return pl.pallas_call(
        body,
        grid=(NB,),
        in_specs=[
            pl.BlockSpec((R, 128), lambda i: (i, 0)),
            pl.BlockSpec((R, 128), lambda i: (i, 0)),
            pl.BlockSpec((R, 128), lambda i: (i, 0)),
            pl.BlockSpec((R, 128), lambda i: (i, 0)),
            pl.BlockSpec((R, 128), lambda i: (i, 0)),
            pl.BlockSpec((R, 128), lambda i: (i, 0)),
            pl.BlockSpec((R, 128), lambda i: (i, 0)),
            pl.BlockSpec((R, G), lambda i: (i, 0)),
            pl.BlockSpec((128, D_HID), lambda i: (0, 0)),
            pl.BlockSpec((128, D_HID), lambda i: (0, 0)),
            pl.BlockSpec((128, D_HID), lambda i: (0, 0)),
            pl.BlockSpec((128, D_HID), lambda i: (0, 0)),
            pl.BlockSpec((1, D_HID), lambda i: (0, 0)),
        ],
        out_specs=[
            pl.BlockSpec((G, D_HID), lambda i: (0, 0)),
            pl.BlockSpec((G, 128), lambda i: (0, 0)),
        ],
        out_shape=[
            jax.ShapeDtypeStruct((G, D_HID), jnp.float32),
            jax.ShapeDtypeStruct((G, 128), jnp.float32),
        ],
    )(sh[0], sh[1], sh[2], sh[3], dega, degb, mt128, pt,
      W2c[0], W2c[1], W2c[2], W2c[3], b2r)


def kernel(x, edge_index, batch, W1, b1, W2, b2):
    m = jax.random.bernoulli(jax.random.key(42), P_KEEP, (N,)).astype(jnp.float32)
    m2 = m[:, None]

    src = edge_index[0]
    dst = edge_index[1]
    pad = E_PAD - E
    src2 = jnp.concatenate([src, jnp.zeros((pad,), jnp.int32)]).reshape(-1, BLK)
    dst2 = jnp.concatenate([dst, jnp.full((pad,), N, jnp.int32)]).reshape(-1, BLK)

    z128 = jnp.zeros((NACC, 128), jnp.float32)
    pt = (batch[:, None] == jnp.arange(G, dtype=jnp.int32)[None, :]).astype(jnp.float32)

    xp0, xp1, mt128 = _prep(x, m2)
    sx0, sx1, dega, degb = _sc_aggregate_l1(xp0, xp1, mt128, src2, dst2, z128)
    h, hp0, hp1, hp2, hp3 = _dense_l1(
        sx0, sx1, dega, degb, mt128, W1[:128], W1[128:], b1[None, :])
    sh = _sc_aggregate_l2(hp0, hp1, hp2, hp3, src2, dst2, z128)
    W2c = [W2[128 * i : 128 * (i + 1)] for i in range(4)]
    pooled, _cnt = _dense_l2_pool(sh, dega, degb, mt128, pt, W2c, b2[None, :])
    e_u = jnp.broadcast_to(pooled[None], (4, G, D_HID))
    return (e_u, h)


# speedup vs baseline: 1.0622x; 1.0622x over previous
import functools

import jax
import jax.numpy as jnp
from jax import lax
from jax.experimental import pallas as pl
from jax.experimental.pallas import tpu as pltpu
from jax.experimental.pallas import tpu_sc as plsc

N = 10000
E = 160000
D_IN = 256
D_HID = 512
G = 64
P_KEEP = 0.7

NC = 2
NS = 16
BLK = 128
E_PAD = 163840
EPT = E_PAD // NS
BPT = EPT // BLK
NACC = 10112
ZR = NACC // NS
DR = NACC // NS

R = 1000
NB = N // R


def _sc_mesh():
    return plsc.VectorSubcoreMesh(
        core_axis_name="c", subcore_axis_name="s", num_cores=NC, num_subcores=NS
    )


NBUF = 2
HB = 40


def _make_pass(z128_h, acc, idx2_v, didx2_v, bufs_v, sem_g, sem_s, r0, d0):
    def do_pass(tab_h, out_h, src2_h, dst2_h, blk_base, nblk):
        pltpu.sync_copy(z128_h.at[pl.ds(r0, ZR)], acc.at[pl.ds(r0, ZR)])

        def wait_g(b, j):
            pltpu.make_async_copy(tab_h.at[idx2_v.at[b]], bufs_v.at[j], sem_g).wait()

        def wait_s(j):
            pltpu.make_async_copy(
                bufs_v.at[j], acc.at[didx2_v.at[0]], sem_s).wait()

        for ph in range(nblk // HB):
            base = blk_base + ph * HB
            pltpu.sync_copy(src2_h.at[pl.ds(base, HB)], idx2_v)
            pltpu.sync_copy(dst2_h.at[pl.ds(base, HB)], didx2_v)
            if ph == 0:
                plsc.subcore_barrier()
            for k in range(NBUF):
                pltpu.async_copy(tab_h.at[idx2_v.at[k]], bufs_v.at[k], sem_g)

            def outer(g, carry):
                for j in range(NBUF):
                    b = g * NBUF + j
                    wait_g(b, j)
                    pltpu.sync_copy(bufs_v.at[j], acc.at[didx2_v.at[b]], add=True)

                    @pl.when(b + NBUF < HB)
                    def _():
                        pltpu.async_copy(
                            tab_h.at[idx2_v.at[b + NBUF]], bufs_v.at[j], sem_g)

                return carry

            lax.fori_loop(0, HB // NBUF, outer, 0)
        plsc.subcore_barrier()
        pltpu.sync_copy(acc.at[pl.ds(d0, DR)], out_h.at[pl.ds(d0, DR)])
        plsc.subcore_barrier()

    return do_pass


_SC_SCRATCH = [
    pltpu.VMEM((HB, BLK), jnp.int32),
    pltpu.VMEM((HB, BLK), jnp.int32),
    pltpu.VMEM((NBUF, BLK, 128), jnp.float32),
    pltpu.VMEM_SHARED((NACC, 128), jnp.float32),
    pltpu.SemaphoreType.DMA,
    pltpu.SemaphoreType.DMA,
]


def _sc_aggregate_l1(xp0, xp1, mt128, src2, dst2, z128):
    @functools.partial(
        pl.kernel,
        out_type=[jax.ShapeDtypeStruct((NACC, 128), jnp.float32)] * 4,
        mesh=_sc_mesh(),
        scratch_types=_SC_SCRATCH,
    )
    def k(xp0_h, xp1_h, mt_h, src2_h, dst2_h, z128_h,
          sx0_h, sx1_h, dega_h, degb_h,
          idx2_v, didx2_v, bufs_v, acc, sem_g, sem_s):
        c = lax.axis_index("c")
        s = lax.axis_index("s")
        do_pass = _make_pass(z128_h, acc, idx2_v, didx2_v, bufs_v, sem_g, sem_s,
                             s * ZR, s * DR)
        halfb = (E_PAD // 2) // BLK

        @pl.when(c == 0)
        def _():
            do_pass(xp0_h, sx0_h, src2_h, dst2_h, s * BPT, BPT)
            do_pass(mt_h, dega_h, src2_h, dst2_h, s * (BPT // 2), BPT // 2)

        @pl.when(c == 1)
        def _():
            do_pass(xp1_h, sx1_h, src2_h, dst2_h, s * BPT, BPT)
            do_pass(mt_h, degb_h, src2_h, dst2_h, halfb + s * (BPT // 2), BPT // 2)

    return k(xp0, xp1, mt128, src2, dst2, z128)


def _sc_aggregate_l2(hp0, hp1, hp2, hp3, src2, dst2, z128):
    @functools.partial(
        pl.kernel,
        out_type=[jax.ShapeDtypeStruct((NACC, 128), jnp.float32)] * 4,
        mesh=_sc_mesh(),
        scratch_types=_SC_SCRATCH,
    )
    def k(hp0_h, hp1_h, hp2_h, hp3_h, src2_h, dst2_h, z128_h,
          sh0_h, sh1_h, sh2_h, sh3_h,
          idx2_v, didx2_v, bufs_v, acc, sem_g, sem_s):
        c = lax.axis_index("c")
        s = lax.axis_index("s")
        do_pass = _make_pass(z128_h, acc, idx2_v, didx2_v, bufs_v, sem_g, sem_s,
                             s * ZR, s * DR)

        @pl.when(c == 0)
        def _():
            do_pass(hp0_h, sh0_h, src2_h, dst2_h, s * BPT, BPT)
            do_pass(hp1_h, sh1_h, src2_h, dst2_h, s * BPT, BPT)

        @pl.when(c == 1)
        def _():
            do_pass(hp2_h, sh2_h, src2_h, dst2_h, s * BPT, BPT)
            do_pass(hp3_h, sh3_h, src2_h, dst2_h, s * BPT, BPT)

    return k(hp0, hp1, hp2, hp3, src2, dst2, z128)


def _prep(x, m2):
    def body(x_ref, m_ref, xp0_ref, xp1_ref, mt_ref):
        mcol = m_ref[...]
        xp = x_ref[...] * mcol
        xp0_ref[...] = xp[:, :128]
        xp1_ref[...] = xp[:, 128:]
        mt_ref[...] = jnp.broadcast_to(mcol, (R, 128))

    return pl.pallas_call(
        body,
        grid=(NB,),
        in_specs=[
            pl.BlockSpec((R, D_IN), lambda i: (i, 0)),
            pl.BlockSpec((R, 1), lambda i: (i, 0)),
        ],
        out_specs=[
            pl.BlockSpec((R, 128), lambda i: (i, 0)),
            pl.BlockSpec((R, 128), lambda i: (i, 0)),
            pl.BlockSpec((R, 128), lambda i: (i, 0)),
        ],
        out_shape=[
            jax.ShapeDtypeStruct((N, 128), jnp.float32),
            jax.ShapeDtypeStruct((N, 128), jnp.float32),
            jax.ShapeDtypeStruct((N, 128), jnp.float32),
        ],
    )(x, m2)


def _dense_l1(sx0, sx1, dega, degb, mt128, W1a, W1b, b1r):
    def body(sx0_ref, sx1_ref, dega_ref, degb_ref, mt_ref, wa_ref, wb_ref, b_ref,
             h_ref, hp0_ref, hp1_ref, hp2_ref, hp3_ref):
        mcol = mt_ref[:, 0:1]
        deg = dega_ref[:, 0:1] + degb_ref[:, 0:1]
        den = jnp.maximum(mcol * deg, 1.0)
        sc = mcol / den
        a0 = sx0_ref[...] * sc
        a1 = sx1_ref[...] * sc
        hh = (
            jnp.dot(a0, wa_ref[...], precision=lax.Precision.HIGHEST,
                    preferred_element_type=jnp.float32)
            + jnp.dot(a1, wb_ref[...], precision=lax.Precision.HIGHEST,
                      preferred_element_type=jnp.float32)
            + b_ref[...]
        )
        hh = jnp.maximum(hh, 0.0)
        h_ref[...] = hh
        hp = hh * mcol
        hp0_ref[...] = hp[:, 0:128]
        hp1_ref[...] = hp[:, 128:256]
        hp2_ref[...] = hp[:, 256:384]
        hp3_ref[...] = hp[:, 384:512]

    return pl.pallas_call(
        body,
        grid=(NB,),
        in_specs=[
            pl.BlockSpec((R, 128), lambda i: (i, 0)),
            pl.BlockSpec((R, 128), lambda i: (i, 0)),
            pl.BlockSpec((R, 128), lambda i: (i, 0)),
            pl.BlockSpec((R, 128), lambda i: (i, 0)),
            pl.BlockSpec((R, 128), lambda i: (i, 0)),
            pl.BlockSpec((128, D_HID), lambda i: (0, 0)),
            pl.BlockSpec((128, D_HID), lambda i: (0, 0)),
            pl.BlockSpec((1, D_HID), lambda i: (0, 0)),
        ],
        out_specs=[
            pl.BlockSpec((R, D_HID), lambda i: (i, 0)),
            pl.BlockSpec((R, 128), lambda i: (i, 0)),
            pl.BlockSpec((R, 128), lambda i: (i, 0)),
            pl.BlockSpec((R, 128), lambda i: (i, 0)),
            pl.BlockSpec((R, 128), lambda i: (i, 0)),
        ],
        out_shape=[
            jax.ShapeDtypeStruct((N, D_HID), jnp.float32),
            jax.ShapeDtypeStruct((N, 128), jnp.float32),
            jax.ShapeDtypeStruct((N, 128), jnp.float32),
            jax.ShapeDtypeStruct((N, 128), jnp.float32),
            jax.ShapeDtypeStruct((N, 128), jnp.float32),
        ],
    )(sx0, sx1, dega, degb, mt128, W1a, W1b, b1r)


def _dense_l2_pool(sh, dega, degb, mt128, pt, W2c, b2r):
    def body(sh0_ref, sh1_ref, sh2_ref, sh3_ref, dega_ref, degb_ref, mt_ref,
             pt_ref, w0_ref, w1_ref, w2_ref, w3_ref, b_ref, pooled_ref, cnt_ref):
        i = pl.program_id(0)

        @pl.when(i == 0)
        def _():
            pooled_ref[...] = jnp.zeros_like(pooled_ref)
            cnt_ref[...] = jnp.zeros_like(cnt_ref)

        mcol = mt_ref[:, 0:1]
        deg = dega_ref[:, 0:1] + degb_ref[:, 0:1]
        den = jnp.maximum(mcol * deg, 1.0)
        sc = mcol / den
        v = b_ref[...] + sum(
            jnp.dot(sref[...] * sc, wref[...], precision=lax.Precision.HIGHEST,
                    preferred_element_type=jnp.float32)
            for sref, wref in ((sh0_ref, w0_ref), (sh1_ref, w1_ref),
                               (sh2_ref, w2_ref), (sh3_ref, w3_ref))
        )
        v = jnp.maximum(v, 0.0)
        ptv = pt_ref[...]
        pooled_ref[...] += lax.dot_general(
            ptv, v, (((0,), (0,)), ((), ())),
            precision=lax.Precision.HIGHEST, preferred_element_type=jnp.float32)
        cnt_ref[...] += lax.dot_general(
            ptv, jnp.ones((R, 128), jnp.float32), (((0,), (0,)), ((), ())),
            precision=lax.Precision.HIGHEST, preferred_element_type=jnp.float32)

        @pl.when(i == NB - 1)
        def _():
            pooled_ref[...] = pooled_ref[...] / jnp.maximum(cnt_ref[:, 0:1], 1.0)

    return pl.pallas_call(
        body,
        grid=(NB,),
        in_specs=[
            pl.BlockSpec((R, 128), lambda i: (i, 0)),
            pl.BlockSpec((R, 128), lambda i: (i, 0)),
            pl.BlockSpec((R, 128), lambda i: (i, 0)),
            pl.BlockSpec((R, 128), lambda i: (i, 0)),
            pl.BlockSpec((R, 128), lambda i: (i, 0)),
            pl.BlockSpec((R, 128), lambda i: (i, 0)),
            pl.BlockSpec((R, 128), lambda i: (i, 0)),
            pl.BlockSpec((R, G), lambda i: (i, 0)),
            pl.BlockSpec((128, D_HID), lambda i: (0, 0)),
            pl.BlockSpec((128, D_HID), lambda i: (0, 0)),
            pl.BlockSpec((128, D_HID), lambda i: (0, 0)),
            pl.BlockSpec((128, D_HID), lambda i: (0, 0)),
            pl.BlockSpec((1, D_HID), lambda i: (0, 0)),
        ],
        out_specs=[
            pl.BlockSpec((G, D_HID), lambda i: (0, 0)),
            pl.BlockSpec((G, 128), lambda i: (0, 0)),
        ],
        out_shape=[
            jax.ShapeDtypeStruct((G, D_HID), jnp.float32),
            jax.ShapeDtypeStruct((G, 128), jnp.float32),
        ],
    )(sh[0], sh[1], sh[2], sh[3], dega, degb, mt128, pt,
      W2c[0], W2c[1], W2c[2], W2c[3], b2r)


def kernel(x, edge_index, batch, W1, b1, W2, b2):
    m = jax.random.bernoulli(jax.random.key(42), P_KEEP, (N,)).astype(jnp.float32)
    m2 = m[:, None]

    src = edge_index[0]
    dst = edge_index[1]
    pad = E_PAD - E
    src2 = jnp.concatenate([src, jnp.zeros((pad,), jnp.int32)]).reshape(-1, BLK)
    dst2 = jnp.concatenate([dst, jnp.full((pad,), N, jnp.int32)]).reshape(-1, BLK)

    z128 = jnp.zeros((NACC, 128), jnp.float32)
    pt = (batch[:, None] == jnp.arange(G, dtype=jnp.int32)[None, :]).astype(jnp.float32)

    xp0, xp1, mt128 = _prep(x, m2)
    sx0, sx1, dega, degb = _sc_aggregate_l1(xp0, xp1, mt128, src2, dst2, z128)
    h, hp0, hp1, hp2, hp3 = _dense_l1(
        sx0, sx1, dega, degb, mt128, W1[:128], W1[128:], b1[None, :])
    sh = _sc_aggregate_l2(hp0, hp1, hp2, hp3, src2, dst2, z128)
    W2c = [W2[128 * i : 128 * (i + 1)] for i in range(4)]
    pooled, _cnt = _dense_l2_pool(sh, dega, degb, mt128, pt, W2c, b2[None, :])
    e_u = jnp.broadcast_to(pooled[None], (4, G, D_HID))
    return (e_u, h)


# speedup vs baseline: 1.1914x; 1.1216x over previous
import functools

import jax
import jax.numpy as jnp
from jax import lax
from jax.experimental import pallas as pl
from jax.experimental.pallas import tpu as pltpu
from jax.experimental.pallas import tpu_sc as plsc

N = 10000
E = 160000
D_IN = 256
D_HID = 512
G = 64
P_KEEP = 0.7

NC = 2
NS = 16
BLK = 128
E_PAD = 163840
EPT = E_PAD // NS
BPT = EPT // BLK
NACC = 10112
ZR = NACC // NS
DR = NACC // NS

R = 1000
NB = N // R


def _sc_mesh():
    return plsc.VectorSubcoreMesh(
        core_axis_name="c", subcore_axis_name="s", num_cores=NC, num_subcores=NS
    )


NBUF = 2
HB = 40


def _make_pass(z128_h, acc, idx2_v, didx2_v, bufs_v, sem_g, sem_s, r0, d0):
    def do_pass(tab_h, out_h, src2_h, dst2_h, blk_base, nblk):
        pltpu.sync_copy(z128_h.at[pl.ds(r0, ZR)], acc.at[pl.ds(r0, ZR)])

        def wait_g(b, j):
            pltpu.make_async_copy(tab_h.at[idx2_v.at[b]], bufs_v.at[j], sem_g).wait()

        def wait_s(j):
            pltpu.make_async_copy(
                bufs_v.at[j], acc.at[didx2_v.at[0]], sem_s).wait()

        for ph in range(nblk // HB):
            base = blk_base + ph * HB
            pltpu.sync_copy(src2_h.at[pl.ds(base, HB)], idx2_v)
            pltpu.sync_copy(dst2_h.at[pl.ds(base, HB)], didx2_v)
            if ph == 0:
                plsc.subcore_barrier()
            for k in range(NBUF):
                pltpu.async_copy(tab_h.at[idx2_v.at[k]], bufs_v.at[k], sem_g)

            def outer(g, carry):
                for j in range(NBUF):
                    b = g * NBUF + j
                    wait_g(b, j)
                    pltpu.sync_copy(bufs_v.at[j], acc.at[didx2_v.at[b]], add=True)

                    @pl.when(b + NBUF < HB)
                    def _():
                        pltpu.async_copy(
                            tab_h.at[idx2_v.at[b + NBUF]], bufs_v.at[j], sem_g)

                return carry

            lax.fori_loop(0, HB // NBUF, outer, 0)
        plsc.subcore_barrier()
        pltpu.sync_copy(acc.at[pl.ds(d0, DR)], out_h.at[pl.ds(d0, DR)])
        plsc.subcore_barrier()

    return do_pass


_SC_SCRATCH = [
    pltpu.VMEM((HB, BLK), jnp.int32),
    pltpu.VMEM((HB, BLK), jnp.int32),
    pltpu.VMEM((NBUF, BLK, 128), jnp.float32),
    pltpu.VMEM_SHARED((NACC, 128), jnp.float32),
    pltpu.SemaphoreType.DMA,
    pltpu.SemaphoreType.DMA,
]


def _sc_aggregate_l1(xp0, xp1, mt128, src2, dst2, z128):
    @functools.partial(
        pl.kernel,
        out_type=[jax.ShapeDtypeStruct((NACC, 128), jnp.float32)] * 4,
        mesh=_sc_mesh(),
        scratch_types=_SC_SCRATCH,
    )
    def k(xp0_h, xp1_h, mt_h, src2_h, dst2_h, z128_h,
          sx0_h, sx1_h, dega_h, degb_h,
          idx2_v, didx2_v, bufs_v, acc, sem_g, sem_s):
        c = lax.axis_index("c")
        s = lax.axis_index("s")
        do_pass = _make_pass(z128_h, acc, idx2_v, didx2_v, bufs_v, sem_g, sem_s,
                             s * ZR, s * DR)
        halfb = (E_PAD // 2) // BLK

        @pl.when(c == 0)
        def _():
            do_pass(xp0_h, sx0_h, src2_h, dst2_h, s * BPT, BPT)
            do_pass(mt_h, dega_h, src2_h, dst2_h, s * (BPT // 2), BPT // 2)

        @pl.when(c == 1)
        def _():
            do_pass(xp1_h, sx1_h, src2_h, dst2_h, s * BPT, BPT)
            do_pass(mt_h, degb_h, src2_h, dst2_h, halfb + s * (BPT // 2), BPT // 2)

    return k(xp0, xp1, mt128, src2, dst2, z128)


def _sc_aggregate_l2(hp0, hp1, hp2, hp3, src2, dst2, z128):
    @functools.partial(
        pl.kernel,
        out_type=[jax.ShapeDtypeStruct((NACC, 128), jnp.float32)] * 4,
        mesh=_sc_mesh(),
        scratch_types=_SC_SCRATCH,
    )
    def k(hp0_h, hp1_h, hp2_h, hp3_h, src2_h, dst2_h, z128_h,
          sh0_h, sh1_h, sh2_h, sh3_h,
          idx2_v, didx2_v, bufs_v, acc, sem_g, sem_s):
        c = lax.axis_index("c")
        s = lax.axis_index("s")
        do_pass = _make_pass(z128_h, acc, idx2_v, didx2_v, bufs_v, sem_g, sem_s,
                             s * ZR, s * DR)

        @pl.when(c == 0)
        def _():
            do_pass(hp0_h, sh0_h, src2_h, dst2_h, s * BPT, BPT)
            do_pass(hp1_h, sh1_h, src2_h, dst2_h, s * BPT, BPT)

        @pl.when(c == 1)
        def _():
            do_pass(hp2_h, sh2_h, src2_h, dst2_h, s * BPT, BPT)
            do_pass(hp3_h, sh3_h, src2_h, dst2_h, s * BPT, BPT)

    return k(hp0, hp1, hp2, hp3, src2, dst2, z128)


def _prep(x, m2):
    def body(x_ref, m_ref, xp0_ref, xp1_ref, mt_ref):
        mcol = m_ref[...]
        xp = x_ref[...] * mcol
        xp0_ref[...] = xp[:, :128]
        xp1_ref[...] = xp[:, 128:]
        mt_ref[...] = jnp.broadcast_to(mcol, (R, 128))

    return pl.pallas_call(
        body,
        grid=(NB,),
        in_specs=[
            pl.BlockSpec((R, D_IN), lambda i: (i, 0)),
            pl.BlockSpec((R, 1), lambda i: (i, 0)),
        ],
        out_specs=[
            pl.BlockSpec((R, 128), lambda i: (i, 0)),
            pl.BlockSpec((R, 128), lambda i: (i, 0)),
            pl.BlockSpec((R, 128), lambda i: (i, 0)),
        ],
        out_shape=[
            jax.ShapeDtypeStruct((N, 128), jnp.float32),
            jax.ShapeDtypeStruct((N, 128), jnp.float32),
            jax.ShapeDtypeStruct((N, 128), jnp.float32),
        ],
    )(x, m2)


def _dense_l1(sx0, sx1, dcol, m2, W1a, W1b, b1r):
    def body(sx0_ref, sx1_ref, d_ref, m_ref, wa_ref, wb_ref, b_ref,
             h_ref, hp0_ref, hp1_ref, hp2_ref, hp3_ref):
        mcol = m_ref[...]
        den = jnp.maximum(mcol * d_ref[...], 1.0)
        sc = mcol / den
        a0 = sx0_ref[...] * sc
        a1 = sx1_ref[...] * sc
        hh = (
            jnp.dot(a0, wa_ref[...], precision=lax.Precision.HIGHEST,
                    preferred_element_type=jnp.float32)
            + jnp.dot(a1, wb_ref[...], precision=lax.Precision.HIGHEST,
                      preferred_element_type=jnp.float32)
            + b_ref[...]
        )
        hh = jnp.maximum(hh, 0.0)
        h_ref[...] = hh
        hp = hh * mcol
        hp0_ref[...] = hp[:, 0:128]
        hp1_ref[...] = hp[:, 128:256]
        hp2_ref[...] = hp[:, 256:384]
        hp3_ref[...] = hp[:, 384:512]

    return pl.pallas_call(
        body,
        grid=(NB,),
        in_specs=[
            pl.BlockSpec((R, 128), lambda i: (i, 0)),
            pl.BlockSpec((R, 128), lambda i: (i, 0)),
            pl.BlockSpec((R, 1), lambda i: (i, 0)),
            pl.BlockSpec((R, 1), lambda i: (i, 0)),
            pl.BlockSpec((128, D_HID), lambda i: (0, 0)),
            pl.BlockSpec((128, D_HID), lambda i: (0, 0)),
            pl.BlockSpec((1, D_HID), lambda i: (0, 0)),
        ],
        out_specs=[
            pl.BlockSpec((R, D_HID), lambda i: (i, 0)),
            pl.BlockSpec((R, 128), lambda i: (i, 0)),
            pl.BlockSpec((R, 128), lambda i: (i, 0)),
            pl.BlockSpec((R, 128), lambda i: (i, 0)),
            pl.BlockSpec((R, 128), lambda i: (i, 0)),
        ],
        out_shape=[
            jax.ShapeDtypeStruct((N, D_HID), jnp.float32),
            jax.ShapeDtypeStruct((N, 128), jnp.float32),
            jax.ShapeDtypeStruct((N, 128), jnp.float32),
            jax.ShapeDtypeStruct((N, 128), jnp.float32),
            jax.ShapeDtypeStruct((N, 128), jnp.float32),
        ],
    )(sx0, sx1, dcol, m2, W1a, W1b, b1r)


def _dense_l2_pool(sh, dcol, m2, pt, W2c, b2r):
    def body(sh0_ref, sh1_ref, sh2_ref, sh3_ref, d_ref, m_ref,
             pt_ref, w0_ref, w1_ref, w2_ref, w3_ref, b_ref, pooled_ref, cnt_ref):
        i = pl.program_id(0)

        @pl.when(i == 0)
        def _():
            pooled_ref[...] = jnp.zeros_like(pooled_ref)
            cnt_ref[...] = jnp.zeros_like(cnt_ref)

        mcol = m_ref[...]
        den = jnp.maximum(mcol * d_ref[...], 1.0)
        sc = mcol / den
        v = b_ref[...] + sum(
            jnp.dot(sref[...] * sc, wref[...], precision=lax.Precision.HIGHEST,
                    preferred_element_type=jnp.float32)
            for sref, wref in ((sh0_ref, w0_ref), (sh1_ref, w1_ref),
                               (sh2_ref, w2_ref), (sh3_ref, w3_ref))
        )
        v = jnp.maximum(v, 0.0)
        ptv = pt_ref[...]
        pooled_ref[...] += lax.dot_general(
            ptv, v, (((0,), (0,)), ((), ())),
            precision=lax.Precision.HIGHEST, preferred_element_type=jnp.float32)
        cnt_ref[...] += lax.dot_general(
            ptv, jnp.ones((R, 128), jnp.float32), (((0,), (0,)), ((), ())),
            precision=lax.Precision.HIGHEST, preferred_element_type=jnp.float32)

        @pl.when(i == NB - 1)
        def _():
            pooled_ref[...] = pooled_ref[...] / jnp.maximum(cnt_ref[:, 0:1], 1.0)

    return pl.pallas_call(
        body,
        grid=(NB,),
        in_specs=[
            pl.BlockSpec((R, 128), lambda i: (i, 0)),
            pl.BlockSpec((R, 128), lambda i: (i, 0)),
            pl.BlockSpec((R, 128), lambda i: (i, 0)),
            pl.BlockSpec((R, 128), lambda i: (i, 0)),
            pl.BlockSpec((R, 1), lambda i: (i, 0)),
            pl.BlockSpec((R, 1), lambda i: (i, 0)),
            pl.BlockSpec((R, G), lambda i: (i, 0)),
            pl.BlockSpec((128, D_HID), lambda i: (0, 0)),
            pl.BlockSpec((128, D_HID), lambda i: (0, 0)),
            pl.BlockSpec((128, D_HID), lambda i: (0, 0)),
            pl.BlockSpec((128, D_HID), lambda i: (0, 0)),
            pl.BlockSpec((1, D_HID), lambda i: (0, 0)),
        ],
        out_specs=[
            pl.BlockSpec((G, D_HID), lambda i: (0, 0)),
            pl.BlockSpec((G, 128), lambda i: (0, 0)),
        ],
        out_shape=[
            jax.ShapeDtypeStruct((G, D_HID), jnp.float32),
            jax.ShapeDtypeStruct((G, 128), jnp.float32),
        ],
    )(sh[0], sh[1], sh[2], sh[3], dcol, m2, pt,
      W2c[0], W2c[1], W2c[2], W2c[3], b2r)


def kernel(x, edge_index, batch, W1, b1, W2, b2):
    m = jax.random.bernoulli(jax.random.key(42), P_KEEP, (N,)).astype(jnp.float32)
    m2 = m[:, None]

    src = edge_index[0]
    dst = edge_index[1]
    pad = E_PAD - E
    src2 = jnp.concatenate([src, jnp.zeros((pad,), jnp.int32)]).reshape(-1, BLK)
    dst2 = jnp.concatenate([dst, jnp.full((pad,), N, jnp.int32)]).reshape(-1, BLK)

    z128 = jnp.zeros((NACC, 128), jnp.float32)
    pt = (batch[:, None] == jnp.arange(G, dtype=jnp.int32)[None, :]).astype(jnp.float32)

    xp0, xp1, mt128 = _prep(x, m2)
    sx0, sx1, dega, degb = _sc_aggregate_l1(xp0, xp1, mt128, src2, dst2, z128)
    dcol = (dega[:N, 0] + degb[:N, 0])[:, None]
    h, hp0, hp1, hp2, hp3 = _dense_l1(
        sx0, sx1, dcol, m2, W1[:128], W1[128:], b1[None, :])
    sh = _sc_aggregate_l2(hp0, hp1, hp2, hp3, src2, dst2, z128)
    W2c = [W2[128 * i : 128 * (i + 1)] for i in range(4)]
    pooled, _cnt = _dense_l2_pool(sh, dcol, m2, pt, W2c, b2[None, :])
    e_u = jnp.broadcast_to(pooled[None], (4, G, D_HID))
    return (e_u, h)
